# Initial kernel scaffold; baseline (speedup 1.0000x reference)
import jax
import jax.numpy as jnp
from jax.experimental import pallas as pl


def kernel(x, edge_attr, node_table, edge_table):
    raise NotImplementedError("write your pallas kernel here")


# speedup vs baseline: 1.1389x; 1.1389x over previous
import functools

import jax
import jax.numpy as jnp
from jax import lax
from jax.experimental import pallas as pl
from jax.experimental.pallas import tpu as pltpu
from jax.experimental.pallas import tpu_sc as plsc

NC = 2
NS = 16
NW = NC * NS
CHUNK = 1000


def _round_up(n, m):
    return (n + m - 1) // m * m


@functools.lru_cache(maxsize=None)
def _build(b_node_pad, b_edge_pad, dim):
    n_w_n = b_node_pad // NW
    n_w_e = b_edge_pad // NW
    mesh = plsc.VectorSubcoreMesh(
        core_axis_name="c", subcore_axis_name="s", num_cores=NC, num_subcores=NS
    )

    @functools.partial(
        pl.kernel,
        mesh=mesh,
        compiler_params=pltpu.CompilerParams(use_tc_tiling_on_sc=False),
        out_type=[
            jax.ShapeDtypeStruct((b_node_pad, dim), jnp.float32),
            jax.ShapeDtypeStruct((b_edge_pad, dim), jnp.float32),
        ],
        scratch_types=[
            pltpu.VMEM((CHUNK,), jnp.int32),
            pltpu.VMEM((CHUNK, dim), jnp.float32),
            pltpu.SemaphoreType.DMA,
        ],
    )
    def emb_kernel(x_hbm, e_hbm, ntab, etab, out_n, out_e, idx_v, rows_v, sem):
        wid = lax.axis_index("s") * NC + lax.axis_index("c")

        def do_chunk(idx_hbm, tab_hbm, out_hbm, off, size):
            pltpu.sync_copy(idx_hbm.at[pl.ds(off, size)], idx_v.at[pl.ds(0, size)])
            pltpu.async_copy(
                tab_hbm.at[idx_v.at[pl.ds(0, size)]],
                rows_v.at[pl.ds(0, size)],
                sem,
            ).wait()
            pltpu.sync_copy(rows_v.at[pl.ds(0, size)], out_hbm.at[pl.ds(off, size)])

        def phase(idx_hbm, tab_hbm, out_hbm, n_w):
            base = wid * n_w
            k_full = n_w // CHUNK
            rem = n_w % CHUNK
            if k_full:
                @pl.loop(0, k_full)
                def _(i):
                    off = pl.multiple_of(base + i * CHUNK, 8)
                    do_chunk(idx_hbm, tab_hbm, out_hbm, off, CHUNK)
            if rem:
                off = pl.multiple_of(base + k_full * CHUNK, 8)
                do_chunk(idx_hbm, tab_hbm, out_hbm, off, rem)

        phase(e_hbm, etab, out_e, n_w_e)
        phase(x_hbm, ntab, out_n, n_w_n)

    return emb_kernel


def kernel(x, edge_attr, node_table, edge_table):
    b_n = x.shape[0]
    b_e = edge_attr.shape[0]
    dim = node_table.shape[1]
    b_n_pad = _round_up(b_n, NW * 8)
    b_e_pad = _round_up(b_e, NW * 8)
    x_i = jnp.pad(x.astype(jnp.int32), (0, b_n_pad - b_n))
    e_i = jnp.pad(edge_attr.astype(jnp.int32), (0, b_e_pad - b_e))
    out_n, out_e = _build(b_n_pad, b_e_pad, dim)(x_i, e_i, node_table, edge_table)
    return (out_n[:b_n], out_e[:b_e])
